# baseline (speedup 1.0000x reference)
import functools

import jax
import jax.numpy as jnp
from jax import lax
from jax.experimental import pallas as pl
from jax.experimental.pallas import tpu as pltpu
from jax.experimental.pallas import tpu_sc as plsc

N = 10000
E = 320000
D = 128
NC = 2
NS = 16
NW = NC * NS
CH = 128
NCHUNK = 80
NCHA = 82
EPW = NCHA * CH
CAP = NW * EPW
NREAL = NCHUNK * CH * NW
NP = 10112
STRIPE = NP // NS

_MESH = dict(core_axis_name="c", subcore_axis_name="s")


def _sc_degree(dst3, ones_rows, zeros16):
    mesh = plsc.VectorSubcoreMesh(**_MESH)

    @functools.partial(
        pl.kernel,
        out_type=jax.ShapeDtypeStruct((NC, NP, 16), jnp.float32),
        mesh=mesh,
        scratch_types=[
            pltpu.VMEM((NCHA, CH), jnp.int32),
            pltpu.VMEM((CH, 16), jnp.float32),
            pltpu.VMEM_SHARED((NP, 16), jnp.float32),
        ],
    )
    def k(dst_hbm, ones_hbm, zero_hbm, out_hbm, dst_v, ones_v, acc):
        c = lax.axis_index("c")
        s = lax.axis_index("s")
        wid = c * NS + s
        base = s * STRIPE
        pltpu.sync_copy(zero_hbm, acc.at[pl.ds(base, STRIPE)])
        pltpu.sync_copy(dst_hbm.at[wid], dst_v)
        pltpu.sync_copy(ones_hbm, ones_v)
        plsc.subcore_barrier()

        @pl.loop(0, NCHUNK)
        def _(j):
            pltpu.sync_copy(ones_v, acc.at[dst_v.at[j]], add=True)

        plsc.subcore_barrier()
        pltpu.sync_copy(acc.at[pl.ds(base, STRIPE)],
                        out_hbm.at[c, pl.ds(base, STRIPE)])

    return k(dst3, ones_rows, zeros16)


HD = D // 2
RB = NP // 8


def _sc_spmm(hs_a, hs_b, src3, dst3, zeros64):
    mesh = plsc.VectorSubcoreMesh(**_MESH)

    @functools.partial(
        pl.kernel,
        out_type=jax.ShapeDtypeStruct((2, NC, NP, HD), jnp.float32),
        mesh=mesh,
        compiler_params=pltpu.CompilerParams(use_tc_tiling_on_sc=False),
        scratch_types=[
            pltpu.VMEM((NCHA, CH), jnp.int32),
            pltpu.VMEM((NCHA, CH), jnp.int32),
            pltpu.VMEM((CH, HD), jnp.float32),
            pltpu.VMEM((CH, HD), jnp.float32),
            pltpu.VMEM_SHARED((NP, HD), jnp.float32),
            pltpu.SemaphoreType.DMA,
            pltpu.SemaphoreType.DMA,
        ],
    )
    def k(hsa_hbm, hsb_hbm, src_hbm, dst_hbm, zero_hbm, out_hbm,
          src_v, dst_v, buf_a, buf_b, acc, sem_a, sem_b):
        c = lax.axis_index("c")
        s = lax.axis_index("s")
        wid = c * NS + s
        base = s * STRIPE
        pltpu.sync_copy(src_hbm.at[wid], src_v)
        pltpu.sync_copy(dst_hbm.at[wid], dst_v)

        for h, hs_hbm in ((0, hsa_hbm), (1, hsb_hbm)):
            pltpu.sync_copy(zero_hbm, acc.at[pl.ds(base, STRIPE)])
            plsc.subcore_barrier()

            pltpu.sync_copy(hs_hbm.at[src_v.at[0]], buf_a)
            pltpu.async_copy(buf_a, acc.at[dst_v.at[0]], sem_a, add=True)

            @pl.loop(1, NCHUNK + 1, step=2)
            def _(j):
                pltpu.sync_copy(hs_hbm.at[src_v.at[j]], buf_b)
                pltpu.async_copy(buf_b, acc.at[dst_v.at[j]], sem_b, add=True)
                pltpu.make_async_copy(buf_a, acc.at[dst_v.at[j - 1]], sem_a).wait()
                pltpu.sync_copy(hs_hbm.at[src_v.at[j + 1]], buf_a)
                pltpu.async_copy(buf_a, acc.at[dst_v.at[j + 1]], sem_a, add=True)
                pltpu.make_async_copy(buf_b, acc.at[dst_v.at[j]], sem_b).wait()

            pltpu.make_async_copy(buf_a, acc.at[dst_v.at[NCHUNK]], sem_a).wait()
            plsc.subcore_barrier()
            pltpu.sync_copy(acc.at[pl.ds(base, STRIPE)],
                            out_hbm.at[h, c, pl.ds(base, STRIPE)])
            plsc.subcore_barrier()

    return k(hs_a, hs_b, src3, dst3, zeros64)


_PREC = lax.Precision.HIGHEST


def _tc_pre(xp, W1, degp):
    def body(x_ref, w_ref, deg_ref, hsa_ref, hsb_ref, dis_ref):
        deg = deg_ref[0, :, 0:1] + deg_ref[1, :, 0:1] + 1.0
        dis = lax.rsqrt(deg)
        h = jnp.dot(x_ref[...], w_ref[...],
                    preferred_element_type=jnp.float32, precision=_PREC)
        hs = h * dis
        hsa_ref[...] = hs[:, :HD]
        hsb_ref[...] = hs[:, HD:]
        dis_ref[...] = dis

    return pl.pallas_call(
        body,
        grid=(NP // RB,),
        in_specs=[
            pl.BlockSpec((RB, D), lambda i: (i, 0)),
            pl.BlockSpec((D, D), lambda i: (0, 0)),
            pl.BlockSpec((NC, RB, 16), lambda i: (0, i, 0)),
        ],
        out_specs=(
            pl.BlockSpec((RB, HD), lambda i: (i, 0)),
            pl.BlockSpec((RB, HD), lambda i: (i, 0)),
            pl.BlockSpec((RB, 1), lambda i: (i, 0)),
        ),
        out_shape=(
            jax.ShapeDtypeStruct((NP, HD), jnp.float32),
            jax.ShapeDtypeStruct((NP, HD), jnp.float32),
            jax.ShapeDtypeStruct((NP, 1), jnp.float32),
        ),
    )(xp, W1, degp)


def _relu_halves(s_ref, hsa_ref, hsb_ref, dis, b_ref, mask_pad):
    ha = dis * (s_ref[0, 0] + s_ref[0, 1] + hsa_ref[...]) + b_ref[:, :HD]
    hb = dis * (s_ref[1, 0] + s_ref[1, 1] + hsb_ref[...]) + b_ref[:, HD:]
    ha = jnp.maximum(ha, 0.0)
    hb = jnp.maximum(hb, 0.0)
    if mask_pad:
        rows = (lax.broadcasted_iota(jnp.int32, (RB, 1), 0)
                + pl.program_id(0) * RB)
        ha = jnp.where(rows < N, ha, 0.0)
        hb = jnp.where(rows < N, hb, 0.0)
    return ha, hb


def _tc_mid(s1, hs1a, hs1b, dis, b1, W2):
    def body(s_ref, hsa_ref, hsb_ref, dis_ref, b_ref, w_ref, outa_ref, outb_ref):
        dis = dis_ref[...]
        ha, hb = _relu_halves(s_ref, hsa_ref, hsb_ref, dis, b_ref, True)
        h2 = (jnp.dot(ha, w_ref[:HD, :],
                      preferred_element_type=jnp.float32, precision=_PREC)
              + jnp.dot(hb, w_ref[HD:, :],
                        preferred_element_type=jnp.float32, precision=_PREC))
        hs2 = dis * h2
        outa_ref[...] = hs2[:, :HD]
        outb_ref[...] = hs2[:, HD:]

    return pl.pallas_call(
        body,
        grid=(NP // RB,),
        in_specs=[
            pl.BlockSpec((2, NC, RB, HD), lambda i: (0, 0, i, 0)),
            pl.BlockSpec((RB, HD), lambda i: (i, 0)),
            pl.BlockSpec((RB, HD), lambda i: (i, 0)),
            pl.BlockSpec((RB, 1), lambda i: (i, 0)),
            pl.BlockSpec((1, D), lambda i: (0, 0)),
            pl.BlockSpec((D, D), lambda i: (0, 0)),
        ],
        out_specs=(
            pl.BlockSpec((RB, HD), lambda i: (i, 0)),
            pl.BlockSpec((RB, HD), lambda i: (i, 0)),
        ),
        out_shape=(
            jax.ShapeDtypeStruct((NP, HD), jnp.float32),
            jax.ShapeDtypeStruct((NP, HD), jnp.float32),
        ),
    )(s1, hs1a, hs1b, dis, b1, W2)


def _tc_post(s2, hs2a, hs2b, dis, b2, Wfc, bfc):
    def body(s_ref, hsa_ref, hsb_ref, dis_ref, b_ref, w_ref, bf_ref, out_ref):
        dis = dis_ref[...]
        ha, hb = _relu_halves(s_ref, hsa_ref, hsb_ref, dis, b_ref, False)
        out_ref[...] = (jnp.dot(ha, w_ref[:HD, :],
                                preferred_element_type=jnp.float32, precision=_PREC)
                        + jnp.dot(hb, w_ref[HD:, :],
                                  preferred_element_type=jnp.float32, precision=_PREC)
                        + bf_ref[...])

    return pl.pallas_call(
        body,
        grid=(NP // RB,),
        in_specs=[
            pl.BlockSpec((2, NC, RB, HD), lambda i: (0, 0, i, 0)),
            pl.BlockSpec((RB, HD), lambda i: (i, 0)),
            pl.BlockSpec((RB, HD), lambda i: (i, 0)),
            pl.BlockSpec((RB, 1), lambda i: (i, 0)),
            pl.BlockSpec((1, D), lambda i: (0, 0)),
            pl.BlockSpec((D, 1), lambda i: (0, 0)),
            pl.BlockSpec((1, 1), lambda i: (0, 0)),
        ],
        out_specs=pl.BlockSpec((RB, 1), lambda i: (i, 0)),
        out_shape=jax.ShapeDtypeStruct((NP, 1), jnp.float32),
    )(s2, hs2a, hs2b, dis, b2, Wfc, bfc)


def kernel(x, edge_index, W1, b1, W2, b2, Wfc, bfc):
    src = edge_index[0].astype(jnp.int32)
    dst = edge_index[1].astype(jnp.int32)
    pad_blk = jnp.full((NW, EPW - E // NW), N, jnp.int32)
    src3 = jnp.concatenate([src.reshape(NW, E // NW), pad_blk], 1).reshape(NW, NCHA, CH)
    dst3 = jnp.concatenate([dst.reshape(NW, E // NW), pad_blk], 1).reshape(NW, NCHA, CH)

    xp = jnp.zeros((NP, D), jnp.float32).at[:N].set(x)
    ones_rows = jnp.ones((CH, 16), jnp.float32)
    zeros16 = jnp.zeros((STRIPE, 16), jnp.float32)
    zeros64 = jnp.zeros((STRIPE, HD), jnp.float32)

    degp = _sc_degree(dst3, ones_rows, zeros16)
    hs1a, hs1b, dis = _tc_pre(xp, W1, degp)
    s1 = _sc_spmm(hs1a, hs1b, src3, dst3, zeros64)
    hs2a, hs2b = _tc_mid(s1, hs1a, hs1b, dis, b1.reshape(1, D), W2)
    s2 = _sc_spmm(hs2a, hs2b, src3, dst3, zeros64)
    outp = _tc_post(s2, hs2a, hs2b, dis, b2.reshape(1, D),
                    Wfc, bfc.reshape(1, 1))
    return outp[:N]


# speedup vs baseline: 1.2451x; 1.2451x over previous
import functools

import jax
import jax.numpy as jnp
from jax import lax
from jax.experimental import pallas as pl
from jax.experimental.pallas import tpu as pltpu
from jax.experimental.pallas import tpu_sc as plsc

N = 10000
E = 320000
D = 128
NC = 2
NS = 16
NW = NC * NS
CH = 224
NCHUNK = 45
NCHA = 46
EPW = NCHA * CH
NP = 10112
STRIPE = NP // NS
RB = NP // 8

_MESH = dict(core_axis_name="c", subcore_axis_name="s")
_NOTILE = pltpu.CompilerParams(use_tc_tiling_on_sc=False)


def _sc_degree(dst3, ones_rows, zeros16):
    mesh = plsc.VectorSubcoreMesh(**_MESH)

    @functools.partial(
        pl.kernel,
        out_type=jax.ShapeDtypeStruct((NC, NP, 16), jnp.float32),
        mesh=mesh,
        compiler_params=_NOTILE,
        scratch_types=[
            pltpu.VMEM((NCHA, CH), jnp.int32),
            pltpu.VMEM((CH, 16), jnp.float32),
            pltpu.VMEM_SHARED((NP, 16), jnp.float32),
        ],
    )
    def k(dst_hbm, ones_hbm, zero_hbm, out_hbm, dst_v, ones_v, acc):
        c = lax.axis_index("c")
        s = lax.axis_index("s")
        wid = c * NS + s
        base = s * STRIPE
        pltpu.sync_copy(zero_hbm, acc.at[pl.ds(base, STRIPE)])
        pltpu.sync_copy(dst_hbm.at[wid], dst_v)
        pltpu.sync_copy(ones_hbm, ones_v)
        plsc.subcore_barrier()

        @pl.loop(0, NCHUNK)
        def _(j):
            pltpu.sync_copy(ones_v, acc.at[dst_v.at[j]], add=True)

        plsc.subcore_barrier()
        pltpu.sync_copy(acc.at[pl.ds(base, STRIPE)],
                        out_hbm.at[c, pl.ds(base, STRIPE)])

    return k(dst3, ones_rows, zeros16)


def _sc_spmm(hs, src3, dst3):
    mesh = plsc.VectorSubcoreMesh(**_MESH)

    @functools.partial(
        pl.kernel,
        out_type=jax.ShapeDtypeStruct((NC, NP, D), jnp.float32),
        mesh=mesh,
        compiler_params=_NOTILE,
        scratch_types=[
            pltpu.VMEM((NCHA, CH), jnp.int32),
            pltpu.VMEM((NCHA, CH), jnp.int32),
            pltpu.VMEM((CH, D), jnp.float32),
            pltpu.VMEM_SHARED((NP, D), jnp.float32),
        ],
    )
    def k(hs_hbm, src_hbm, dst_hbm, out_hbm,
          src_v, dst_v, rows_v, acc):
        c = lax.axis_index("c")
        s = lax.axis_index("s")
        wid = c * NS + s
        base = s * STRIPE
        pltpu.sync_copy(src_hbm.at[wid], src_v)
        pltpu.sync_copy(dst_hbm.at[wid], dst_v)
        pltpu.sync_copy(hs_hbm.at[src_v.at[NCHUNK]], rows_v)
        pltpu.sync_copy(rows_v, acc.at[pl.ds(base, CH)])
        pltpu.sync_copy(rows_v, acc.at[pl.ds(base + CH, CH)])
        pltpu.sync_copy(rows_v.at[pl.ds(0, STRIPE - 2 * CH)],
                        acc.at[pl.ds(base + 2 * CH, STRIPE - 2 * CH)])
        plsc.subcore_barrier()

        @pl.loop(0, NCHUNK)
        def _(j):
            pltpu.sync_copy(hs_hbm.at[src_v.at[j]], rows_v)
            pltpu.sync_copy(rows_v, acc.at[dst_v.at[j]], add=True)

        plsc.subcore_barrier()
        pltpu.sync_copy(acc.at[pl.ds(base, STRIPE)],
                        out_hbm.at[c, pl.ds(base, STRIPE)])

    return k(hs, src3, dst3)


_PREC = lax.Precision.HIGHEST


def _tc_pre(xp, W1, degp):
    def body(x_ref, w_ref, deg_ref, hs_ref, dis_ref):
        deg = deg_ref[0, :, 0:1] + deg_ref[1, :, 0:1] + 1.0
        dis = lax.rsqrt(deg)
        h = jnp.dot(x_ref[...], w_ref[...],
                    preferred_element_type=jnp.float32, precision=_PREC)
        hs_ref[...] = h * dis
        dis_ref[...] = dis

    return pl.pallas_call(
        body,
        grid=(NP // RB,),
        in_specs=[
            pl.BlockSpec((RB, D), lambda i: (i, 0)),
            pl.BlockSpec((D, D), lambda i: (0, 0)),
            pl.BlockSpec((NC, RB, 16), lambda i: (0, i, 0)),
        ],
        out_specs=(
            pl.BlockSpec((RB, D), lambda i: (i, 0)),
            pl.BlockSpec((RB, 1), lambda i: (i, 0)),
        ),
        out_shape=(
            jax.ShapeDtypeStruct((NP, D), jnp.float32),
            jax.ShapeDtypeStruct((NP, 1), jnp.float32),
        ),
    )(xp, W1, degp)


def _tc_mid(s1, hs1, dis, b1, W2):
    def body(s_ref, hs_ref, dis_ref, b_ref, w_ref, out_ref):
        dis = dis_ref[...]
        h = dis * (s_ref[0] + s_ref[1] + hs_ref[...]) + b_ref[...]
        h = jnp.maximum(h, 0.0)
        rows = (lax.broadcasted_iota(jnp.int32, (RB, 1), 0)
                + pl.program_id(0) * RB)
        h = jnp.where(rows < N, h, 0.0)
        out_ref[...] = dis * jnp.dot(h, w_ref[...],
                                     preferred_element_type=jnp.float32,
                                     precision=_PREC)

    return pl.pallas_call(
        body,
        grid=(NP // RB,),
        in_specs=[
            pl.BlockSpec((NC, RB, D), lambda i: (0, i, 0)),
            pl.BlockSpec((RB, D), lambda i: (i, 0)),
            pl.BlockSpec((RB, 1), lambda i: (i, 0)),
            pl.BlockSpec((1, D), lambda i: (0, 0)),
            pl.BlockSpec((D, D), lambda i: (0, 0)),
        ],
        out_specs=pl.BlockSpec((RB, D), lambda i: (i, 0)),
        out_shape=jax.ShapeDtypeStruct((NP, D), jnp.float32),
    )(s1, hs1, dis, b1, W2)


def _tc_post(s2, hs2, dis, b2, Wfc, bfc):
    def body(s_ref, hs_ref, dis_ref, b_ref, w_ref, bf_ref, out_ref):
        dis = dis_ref[...]
        h = dis * (s_ref[0] + s_ref[1] + hs_ref[...]) + b_ref[...]
        h = jnp.maximum(h, 0.0)
        out_ref[...] = jnp.dot(h, w_ref[...],
                               preferred_element_type=jnp.float32,
                               precision=_PREC) + bf_ref[...]

    return pl.pallas_call(
        body,
        grid=(NP // RB,),
        in_specs=[
            pl.BlockSpec((NC, RB, D), lambda i: (0, i, 0)),
            pl.BlockSpec((RB, D), lambda i: (i, 0)),
            pl.BlockSpec((RB, 1), lambda i: (i, 0)),
            pl.BlockSpec((1, D), lambda i: (0, 0)),
            pl.BlockSpec((D, 1), lambda i: (0, 0)),
            pl.BlockSpec((1, 1), lambda i: (0, 0)),
        ],
        out_specs=pl.BlockSpec((RB, 1), lambda i: (i, 0)),
        out_shape=jax.ShapeDtypeStruct((NP, 1), jnp.float32),
    )(s2, hs2, dis, b2, Wfc, bfc)


def kernel(x, edge_index, W1, b1, W2, b2, Wfc, bfc):
    src = edge_index[0].astype(jnp.int32)
    dst = edge_index[1].astype(jnp.int32)
    pad_blk = jnp.full((NW, EPW - E // NW), N, jnp.int32)
    src3 = jnp.concatenate([src.reshape(NW, E // NW), pad_blk], 1).reshape(NW, NCHA, CH)
    dst3 = jnp.concatenate([dst.reshape(NW, E // NW), pad_blk], 1).reshape(NW, NCHA, CH)

    xp = jnp.zeros((NP, D), jnp.float32).at[:N].set(x)
    ones_rows = jnp.ones((CH, 16), jnp.float32)
    zeros16 = jnp.zeros((STRIPE, 16), jnp.float32)

    degp = _sc_degree(dst3, ones_rows, zeros16)
    hs1, dis = _tc_pre(xp, W1, degp)
    s1 = _sc_spmm(hs1, src3, dst3)
    hs2 = _tc_mid(s1, hs1, dis, b1.reshape(1, D), W2)
    s2 = _sc_spmm(hs2, src3, dst3)
    outp = _tc_post(s2, hs2, dis, b2.reshape(1, D),
                    Wfc, bfc.reshape(1, 1))
    return outp[:N]


# speedup vs baseline: 1.3819x; 1.1098x over previous
import functools

import jax
import jax.numpy as jnp
from jax import lax
from jax.experimental import pallas as pl
from jax.experimental.pallas import tpu as pltpu
from jax.experimental.pallas import tpu_sc as plsc

N = 10000
E = 320000
D = 128
NC = 2
NS = 16
NW = NC * NS
CH = 128
NCHUNK = 79
NCHA = 80
EPW = NCHA * CH
NP = 10112
STRIPE = NP // NS
RB = NP // 8

_MESH = dict(core_axis_name="c", subcore_axis_name="s")
_NOTILE = pltpu.CompilerParams(use_tc_tiling_on_sc=False)


def _sc_degree(dst3, ones_rows, zeros16):
    mesh = plsc.VectorSubcoreMesh(**_MESH)

    @functools.partial(
        pl.kernel,
        out_type=jax.ShapeDtypeStruct((NC, NP, 16), jnp.float32),
        mesh=mesh,
        compiler_params=_NOTILE,
        scratch_types=[
            pltpu.VMEM((NCHA, CH), jnp.int32),
            pltpu.VMEM((CH, 16), jnp.float32),
            pltpu.VMEM_SHARED((NP, 16), jnp.float32),
        ],
    )
    def k(dst_hbm, ones_hbm, zero_hbm, out_hbm, dst_v, ones_v, acc):
        c = lax.axis_index("c")
        s = lax.axis_index("s")
        wid = c * NS + s
        base = s * STRIPE
        pltpu.sync_copy(zero_hbm, acc.at[pl.ds(base, STRIPE)])
        pltpu.sync_copy(dst_hbm.at[wid], dst_v)
        pltpu.sync_copy(ones_hbm, ones_v)
        plsc.subcore_barrier()

        @pl.loop(0, NCHUNK)
        def _(j):
            pltpu.sync_copy(ones_v, acc.at[dst_v.at[j]], add=True)

        plsc.subcore_barrier()
        pltpu.sync_copy(acc.at[pl.ds(base, STRIPE)],
                        out_hbm.at[c, pl.ds(base, STRIPE)])

    return k(dst3, ones_rows, zeros16)


def _sc_spmm(hs, src3, dst3):
    mesh = plsc.VectorSubcoreMesh(**_MESH)

    @functools.partial(
        pl.kernel,
        out_type=jax.ShapeDtypeStruct((NC, NP, D), jnp.float32),
        mesh=mesh,
        compiler_params=_NOTILE,
        scratch_types=[
            pltpu.VMEM((NCHA, CH), jnp.int32),
            pltpu.VMEM((NCHA, CH), jnp.int32),
            pltpu.VMEM((CH, D), jnp.float32),
            pltpu.VMEM_SHARED((NP, D), jnp.float32),
        ],
    )
    def k(hs_hbm, src_hbm, dst_hbm, out_hbm,
          src_v, dst_v, rows_v, acc):
        c = lax.axis_index("c")
        s = lax.axis_index("s")
        wid = c * NS + s
        base = s * STRIPE
        pltpu.sync_copy(src_hbm.at[wid], src_v)
        pltpu.sync_copy(dst_hbm.at[wid], dst_v)
        pltpu.sync_copy(hs_hbm.at[src_v.at[NCHUNK]], rows_v)
        for kk in range(STRIPE // CH):
            pltpu.sync_copy(rows_v, acc.at[pl.ds(base + kk * CH, CH)])
        if STRIPE % CH:
            pltpu.sync_copy(rows_v.at[pl.ds(0, STRIPE % CH)],
                            acc.at[pl.ds(base + (STRIPE // CH) * CH, STRIPE % CH)])
        plsc.subcore_barrier()

        @pl.loop(0, NCHUNK)
        def _(j):
            pltpu.sync_copy(hs_hbm.at[src_v.at[j]], rows_v)
            pltpu.sync_copy(rows_v, acc.at[dst_v.at[j]], add=True)

        plsc.subcore_barrier()
        pltpu.sync_copy(acc.at[pl.ds(base, STRIPE)],
                        out_hbm.at[c, pl.ds(base, STRIPE)])

    return k(hs, src3, dst3)


_PREC = lax.Precision.HIGHEST


def _tc_pre(xp, W1, degp):
    def body(x_ref, w_ref, deg_ref, hs_ref, dis_ref):
        deg = deg_ref[0, :, 0:1] + deg_ref[1, :, 0:1] + 1.0
        dis = lax.rsqrt(deg)
        h = jnp.dot(x_ref[...], w_ref[...],
                    preferred_element_type=jnp.float32, precision=_PREC)
        hs_ref[...] = h * dis
        dis_ref[...] = dis

    return pl.pallas_call(
        body,
        grid=(NP // RB,),
        in_specs=[
            pl.BlockSpec((RB, D), lambda i: (i, 0)),
            pl.BlockSpec((D, D), lambda i: (0, 0)),
            pl.BlockSpec((NC, RB, 16), lambda i: (0, i, 0)),
        ],
        out_specs=(
            pl.BlockSpec((RB, D), lambda i: (i, 0)),
            pl.BlockSpec((RB, 1), lambda i: (i, 0)),
        ),
        out_shape=(
            jax.ShapeDtypeStruct((NP, D), jnp.float32),
            jax.ShapeDtypeStruct((NP, 1), jnp.float32),
        ),
    )(xp, W1, degp)


def _tc_mid(s1, hs1, dis, b1, W2):
    def body(s_ref, hs_ref, dis_ref, b_ref, w_ref, out_ref):
        dis = dis_ref[...]
        h = dis * (s_ref[0] + s_ref[1] + hs_ref[...]) + b_ref[...]
        h = jnp.maximum(h, 0.0)
        rows = (lax.broadcasted_iota(jnp.int32, (RB, 1), 0)
                + pl.program_id(0) * RB)
        h = jnp.where(rows < N, h, 0.0)
        out_ref[...] = dis * jnp.dot(h, w_ref[...],
                                     preferred_element_type=jnp.float32,
                                     precision=_PREC)

    return pl.pallas_call(
        body,
        grid=(NP // RB,),
        in_specs=[
            pl.BlockSpec((NC, RB, D), lambda i: (0, i, 0)),
            pl.BlockSpec((RB, D), lambda i: (i, 0)),
            pl.BlockSpec((RB, 1), lambda i: (i, 0)),
            pl.BlockSpec((1, D), lambda i: (0, 0)),
            pl.BlockSpec((D, D), lambda i: (0, 0)),
        ],
        out_specs=pl.BlockSpec((RB, D), lambda i: (i, 0)),
        out_shape=jax.ShapeDtypeStruct((NP, D), jnp.float32),
    )(s1, hs1, dis, b1, W2)


def _tc_post(s2, hs2, dis, b2, Wfc, bfc):
    def body(s_ref, hs_ref, dis_ref, b_ref, w_ref, bf_ref, out_ref):
        dis = dis_ref[...]
        h = dis * (s_ref[0] + s_ref[1] + hs_ref[...]) + b_ref[...]
        h = jnp.maximum(h, 0.0)
        out_ref[...] = jnp.dot(h, w_ref[...],
                               preferred_element_type=jnp.float32,
                               precision=_PREC) + bf_ref[...]

    return pl.pallas_call(
        body,
        grid=(NP // RB,),
        in_specs=[
            pl.BlockSpec((NC, RB, D), lambda i: (0, i, 0)),
            pl.BlockSpec((RB, D), lambda i: (i, 0)),
            pl.BlockSpec((RB, 1), lambda i: (i, 0)),
            pl.BlockSpec((1, D), lambda i: (0, 0)),
            pl.BlockSpec((D, 1), lambda i: (0, 0)),
            pl.BlockSpec((1, 1), lambda i: (0, 0)),
        ],
        out_specs=pl.BlockSpec((RB, 1), lambda i: (i, 0)),
        out_shape=jax.ShapeDtypeStruct((NP, 1), jnp.float32),
    )(s2, hs2, dis, b2, Wfc, bfc)


def kernel(x, edge_index, W1, b1, W2, b2, Wfc, bfc):
    src = edge_index[0].astype(jnp.int32)
    dst = edge_index[1].astype(jnp.int32)
    pad_blk = jnp.full((NW, EPW - E // NW), N, jnp.int32)
    src3 = jnp.concatenate([src.reshape(NW, E // NW), pad_blk], 1).reshape(NW, NCHA, CH)
    dst3 = jnp.concatenate([dst.reshape(NW, E // NW), pad_blk], 1).reshape(NW, NCHA, CH)

    xp = jnp.zeros((NP, D), jnp.float32).at[:N].set(x)
    ones_rows = jnp.ones((CH, 16), jnp.float32)
    zeros16 = jnp.zeros((STRIPE, 16), jnp.float32)

    degp = _sc_degree(dst3, ones_rows, zeros16)
    hs1, dis = _tc_pre(xp, W1, degp)
    s1 = _sc_spmm(hs1, src3, dst3)
    hs2 = _tc_mid(s1, hs1, dis, b1.reshape(1, D), W2)
    s2 = _sc_spmm(hs2, src3, dst3)
    outp = _tc_post(s2, hs2, dis, b2.reshape(1, D),
                    Wfc, bfc.reshape(1, 1))
    return outp[:N]


# speedup vs baseline: 1.9544x; 1.4143x over previous
import functools

import jax
import jax.numpy as jnp
from jax import lax
from jax.experimental import pallas as pl
from jax.experimental.pallas import tpu as pltpu
from jax.experimental.pallas import tpu_sc as plsc

N = 10000
E = 320000
D = 128
NC = 2
NS = 16
NW = NC * NS
CH = 128
NCHUNK = 79
EPW = NCHUNK * CH
NP = 10112
STRIPE = NP // NS

_MESH = dict(core_axis_name="c", subcore_axis_name="s")


def _sc_degree(dst3, ones_rows, zeros16):
    mesh = plsc.VectorSubcoreMesh(**_MESH)

    @functools.partial(
        pl.kernel,
        out_type=jax.ShapeDtypeStruct((NC, NP, 16), jnp.float32),
        mesh=mesh,
        scratch_types=[
            pltpu.VMEM((NCHUNK, CH), jnp.int32),
            pltpu.VMEM((CH, 16), jnp.float32),
            pltpu.VMEM_SHARED((NP, 16), jnp.float32),
        ],
    )
    def k(dst_hbm, ones_hbm, zero_hbm, out_hbm, dst_v, ones_v, acc):
        c = lax.axis_index("c")
        s = lax.axis_index("s")
        wid = c * NS + s
        base = s * STRIPE
        pltpu.sync_copy(zero_hbm, acc.at[pl.ds(base, STRIPE)])
        pltpu.sync_copy(dst_hbm.at[wid], dst_v)
        pltpu.sync_copy(ones_hbm, ones_v)
        plsc.subcore_barrier()

        @pl.loop(0, NCHUNK)
        def _(j):
            pltpu.sync_copy(ones_v, acc.at[dst_v.at[j]], add=True)

        plsc.subcore_barrier()
        pltpu.sync_copy(acc.at[pl.ds(base, STRIPE)],
                        out_hbm.at[c, pl.ds(base, STRIPE)])

    return k(dst3, ones_rows, zeros16)


def _sc_spmm(hs, src3, dst3, zeros128):
    mesh = plsc.VectorSubcoreMesh(**_MESH)

    @functools.partial(
        pl.kernel,
        out_type=jax.ShapeDtypeStruct((NC, NP, D), jnp.float32),
        mesh=mesh,
        scratch_types=[
            pltpu.VMEM((NCHUNK, CH), jnp.int32),
            pltpu.VMEM((NCHUNK, CH), jnp.int32),
            pltpu.VMEM((CH, D), jnp.float32),
            pltpu.VMEM_SHARED((NP, D), jnp.float32),
        ],
    )
    def k(hs_hbm, src_hbm, dst_hbm, zero_hbm, out_hbm,
          src_v, dst_v, rows_v, acc):
        c = lax.axis_index("c")
        s = lax.axis_index("s")
        wid = c * NS + s
        base = s * STRIPE
        pltpu.sync_copy(zero_hbm, acc.at[pl.ds(base, STRIPE)])
        pltpu.sync_copy(src_hbm.at[wid], src_v)
        pltpu.sync_copy(dst_hbm.at[wid], dst_v)
        plsc.subcore_barrier()

        @pl.loop(0, NCHUNK)
        def _(j):
            pltpu.sync_copy(hs_hbm.at[src_v.at[j]], rows_v)
            pltpu.sync_copy(rows_v, acc.at[dst_v.at[j]], add=True)

        plsc.subcore_barrier()
        pltpu.sync_copy(acc.at[pl.ds(base, STRIPE)],
                        out_hbm.at[c, pl.ds(base, STRIPE)])

    return k(hs, src3, dst3, zeros128)


_PREC = lax.Precision.HIGHEST


def _tc_matmul(xp, W1):
    def body(x_ref, w_ref, out_ref):
        out_ref[...] = jnp.dot(x_ref[...], w_ref[...],
                               preferred_element_type=jnp.float32,
                               precision=_PREC)

    return pl.pallas_call(
        body,
        out_shape=jax.ShapeDtypeStruct((NP, D), jnp.float32),
    )(xp, W1)


def _tc_scale(h1, degp):
    def body(h_ref, deg_ref, hs_ref, dis_ref):
        deg = deg_ref[0, :, 0:1] + deg_ref[1, :, 0:1] + 1.0
        dis = lax.rsqrt(deg)
        hs_ref[...] = h_ref[...] * dis
        dis_ref[...] = dis

    return pl.pallas_call(
        body,
        out_shape=(
            jax.ShapeDtypeStruct((NP, D), jnp.float32),
            jax.ShapeDtypeStruct((NP, 1), jnp.float32),
        ),
    )(h1, degp)


def _tc_mid(s1, hs1, dis, b1, W2):
    def body(s_ref, hs_ref, dis_ref, b_ref, w_ref, out_ref):
        dis = dis_ref[...]
        h = dis * (s_ref[0] + s_ref[1] + hs_ref[...]) + b_ref[...]
        h = jnp.maximum(h, 0.0)
        rows = lax.broadcasted_iota(jnp.int32, (NP, 1), 0)
        h = jnp.where(rows < N, h, 0.0)
        out_ref[...] = dis * jnp.dot(h, w_ref[...],
                                     preferred_element_type=jnp.float32,
                                     precision=_PREC)

    return pl.pallas_call(
        body,
        out_shape=jax.ShapeDtypeStruct((NP, D), jnp.float32),
    )(s1, hs1, dis, b1, W2)


def _tc_post(s2, hs2, dis, b2, Wfc, bfc):
    def body(s_ref, hs_ref, dis_ref, b_ref, w_ref, bf_ref, out_ref):
        dis = dis_ref[...]
        h = dis * (s_ref[0] + s_ref[1] + hs_ref[...]) + b_ref[...]
        h = jnp.maximum(h, 0.0)
        out_ref[...] = jnp.dot(h, w_ref[...],
                               preferred_element_type=jnp.float32,
                               precision=_PREC) + bf_ref[...]

    return pl.pallas_call(
        body,
        out_shape=jax.ShapeDtypeStruct((NP, 1), jnp.float32),
    )(s2, hs2, dis, b2, Wfc, bfc)


def kernel(x, edge_index, W1, b1, W2, b2, Wfc, bfc):
    src = edge_index[0].astype(jnp.int32)
    dst = edge_index[1].astype(jnp.int32)
    pad_blk = jnp.full((NW, EPW - E // NW), N, jnp.int32)
    src3 = jnp.concatenate([src.reshape(NW, E // NW), pad_blk], 1).reshape(NW, NCHUNK, CH)
    dst3 = jnp.concatenate([dst.reshape(NW, E // NW), pad_blk], 1).reshape(NW, NCHUNK, CH)

    xp = jnp.zeros((NP, D), jnp.float32).at[:N].set(x)
    ones_rows = jnp.ones((CH, 16), jnp.float32)
    zeros16 = jnp.zeros((STRIPE, 16), jnp.float32)
    zeros128 = jnp.zeros((STRIPE, D), jnp.float32)

    h1 = _tc_matmul(xp, W1)
    degp = _sc_degree(dst3, ones_rows, zeros16)
    hs1, dis = _tc_scale(h1, degp)
    s1 = _sc_spmm(hs1, src3, dst3, zeros128)
    hs2 = _tc_mid(s1, hs1, dis, b1.reshape(1, D), W2)
    s2 = _sc_spmm(hs2, src3, dst3, zeros128)
    outp = _tc_post(s2, hs2, dis, b2.reshape(1, D),
                    Wfc, bfc.reshape(1, 1))
    return outp[:N]
